# Initial kernel scaffold; baseline (speedup 1.0000x reference)
import jax
import jax.numpy as jnp
from jax.experimental import pallas as pl


def kernel(x, batch):
    raise NotImplementedError("write your pallas kernel here")


# speedup vs baseline: 4.0133x; 4.0133x over previous
import jax
import jax.numpy as jnp
from jax import lax
from jax.experimental import pallas as pl
from jax.experimental.pallas import tpu as pltpu
from jax.experimental.pallas import tpu_sc as plsc

N = 100000
D = 128
G = 256

NC = 2
NS = 16
L = 16
DH = D // NC
CH = 128

NFULL = N // CH
TAIL = N - NFULL * CH
TAIL_OFF = NFULL * CH
CPT = 49
ACC_ROWS = G + NS
ZR = ACC_ROWS // NS
GPT = G // NS


def _seg_mean_kernel(x_hbm, batch_hbm, out_hbm,
                     acc_sh, cnt_sh, rows_v, ids_v, ones_v, cnt_v):
  tid = lax.axis_index("s")
  cid = lax.axis_index("c")
  c0 = cid * DH
  dummy = (G + tid).astype(jnp.int32)

  def zero_body(r, _):
    for k in range(DH // L):
      rows_v[r, pl.ds(k * L, L)] = jnp.zeros((L,), jnp.float32)
    ones_v[r, pl.ds(0, L)] = jnp.zeros((L,), jnp.float32)
    return 0
  lax.fori_loop(0, ZR, zero_body, 0)
  pltpu.sync_copy(rows_v.at[pl.ds(0, ZR)], acc_sh.at[pl.ds(tid * ZR, ZR)])
  pltpu.sync_copy(ones_v.at[pl.ds(0, ZR)], cnt_sh.at[pl.ds(tid * ZR, ZR)])

  def ones_body(r, _):
    ones_v[r, pl.ds(0, L)] = jnp.ones((L,), jnp.float32)
    return 0
  lax.fori_loop(0, CH, ones_body, 0)

  plsc.subcore_barrier()

  chunk0 = tid * CPT
  n_chunks = jnp.where(tid < NS - 1, CPT, NFULL - (NS - 1) * CPT)

  def chunk_body(k, _):
    off = (chunk0 + k) * CH
    pltpu.sync_copy(batch_hbm.at[pl.ds(off, CH)], ids_v.at[0])
    pltpu.sync_copy(x_hbm.at[pl.ds(off, CH), pl.ds(c0, DH)], rows_v)
    pltpu.sync_copy(rows_v, acc_sh.at[ids_v.at[0]], add=True)
    pltpu.sync_copy(ones_v, cnt_sh.at[ids_v.at[0]], add=True)
    return 0
  lax.fori_loop(0, n_chunks, chunk_body, 0)

  @pl.when(tid == NS - 1)
  def _tail():
    for r in range(CH // L):
      ids_v[0, pl.ds(r * L, L)] = jnp.full((L,), 0, jnp.int32) + dummy
    pltpu.sync_copy(batch_hbm.at[pl.ds(TAIL_OFF, TAIL)],
                    ids_v.at[0, pl.ds(0, TAIL)])
    pltpu.sync_copy(x_hbm.at[pl.ds(TAIL_OFF, TAIL), pl.ds(c0, DH)],
                    rows_v.at[pl.ds(0, TAIL)])
    pltpu.sync_copy(rows_v, acc_sh.at[ids_v.at[0]], add=True)
    pltpu.sync_copy(ones_v, cnt_sh.at[ids_v.at[0]], add=True)

  plsc.subcore_barrier()

  g0 = tid * GPT
  pltpu.sync_copy(acc_sh.at[pl.ds(g0, GPT)], rows_v.at[pl.ds(0, GPT)])
  pltpu.sync_copy(cnt_sh.at[pl.ds(g0, GPT)], cnt_v)

  def div_body(r, _):
    cnt = cnt_v[r, pl.ds(0, L)]
    for k in range(DH // L):
      rows_v[r, pl.ds(k * L, L)] = rows_v[r, pl.ds(k * L, L)] / cnt
    return 0
  lax.fori_loop(0, GPT, div_body, 0)

  pltpu.sync_copy(rows_v.at[pl.ds(0, GPT)],
                  out_hbm.at[pl.ds(g0, GPT), pl.ds(c0, DH)])


def kernel(x, batch):
  mesh = plsc.VectorSubcoreMesh(core_axis_name="c", subcore_axis_name="s")
  return pl.kernel(
      _seg_mean_kernel,
      out_type=jax.ShapeDtypeStruct((G, D), jnp.float32),
      mesh=mesh,
      scratch_types=[
          pltpu.VMEM_SHARED((ACC_ROWS, DH), jnp.float32),
          pltpu.VMEM_SHARED((ACC_ROWS, L), jnp.float32),
          pltpu.VMEM((CH, DH), jnp.float32),
          pltpu.VMEM((1, CH), jnp.int32),
          pltpu.VMEM((CH, L), jnp.float32),
          pltpu.VMEM((GPT, L), jnp.float32),
      ],
      compiler_params=pltpu.CompilerParams(use_tc_tiling_on_sc=False),
  )(x, batch)


# speedup vs baseline: 5.9974x; 1.4944x over previous
import jax
import jax.numpy as jnp
from jax import lax
from jax.experimental import pallas as pl
from jax.experimental.pallas import tpu as pltpu
from jax.experimental.pallas import tpu_sc as plsc

N = 100000
D = 128
G = 256

NC = 2
NS = 16
L = 16
DH = D // NC
CH = 128

NFULL = N // CH
TAIL = N - NFULL * CH
TAIL_OFF = NFULL * CH
CPT = 49
ACC_ROWS = G + NS
ZR = ACC_ROWS // NS
GPT = G // NS


def _seg_mean_kernel(x_hbm, batch_hbm, out_hbm,
                     acc_sh, cnt_sh, rows_v, ids_v, ones_v, cnt_v,
                     sem0, sem1):
  tid = lax.axis_index("s")
  cid = lax.axis_index("c")
  c0 = cid * DH
  dummy = (G + tid).astype(jnp.int32)
  sems = (sem0, sem1)

  def zero_body(r, _):
    for k in range(DH // L):
      rows_v[0, r, pl.ds(k * L, L)] = jnp.zeros((L,), jnp.float32)
    ones_v[r, pl.ds(0, L)] = jnp.zeros((L,), jnp.float32)
    return 0
  lax.fori_loop(0, ZR, zero_body, 0)
  pltpu.sync_copy(rows_v.at[0, pl.ds(0, ZR)], acc_sh.at[pl.ds(tid * ZR, ZR)])
  pltpu.sync_copy(ones_v.at[pl.ds(0, ZR)], cnt_sh.at[pl.ds(tid * ZR, ZR)])

  def ones_body(r, _):
    ones_v[r, pl.ds(0, L)] = jnp.ones((L,), jnp.float32)
    return 0
  lax.fori_loop(0, CH, ones_body, 0)

  plsc.subcore_barrier()

  chunk0 = tid * CPT
  n_chunks = jnp.where(tid < NS - 1, CPT, NFULL - (NS - 1) * CPT)

  def load_copies(k, b):
    off = (chunk0 + k) * CH
    return (
        pltpu.make_async_copy(batch_hbm.at[pl.ds(off, CH)], ids_v.at[b],
                              sems[b]),
        pltpu.make_async_copy(x_hbm.at[pl.ds(off, CH), pl.ds(c0, DH)],
                              rows_v.at[b], sems[b]),
    )

  def start_load(k, b):
    for cp in load_copies(k, b):
      cp.start()

  def wait_load(k, b):
    for cp in load_copies(k, b):
      cp.wait()

  start_load(0, 0)
  start_load(1, 1)

  def pair_body(i, _):
    for b in range(2):
      k = 2 * i + b

      @pl.when(k < n_chunks)
      def _process():
        wait_load(k, b)
        pltpu.sync_copy(rows_v.at[b], acc_sh.at[ids_v.at[b]], add=True)
        pltpu.sync_copy(ones_v, cnt_sh.at[ids_v.at[b]], add=True)

        @pl.when(k + 2 < n_chunks)
        def _prefetch():
          start_load(k + 2, b)
    return 0
  lax.fori_loop(0, (n_chunks + 1) // 2, pair_body, 0)

  @pl.when(tid == NS - 1)
  def _tail():
    for r in range(CH // L):
      ids_v[0, pl.ds(r * L, L)] = jnp.full((L,), 0, jnp.int32) + dummy
    pltpu.sync_copy(batch_hbm.at[pl.ds(TAIL_OFF, TAIL)],
                    ids_v.at[0, pl.ds(0, TAIL)])
    pltpu.sync_copy(x_hbm.at[pl.ds(TAIL_OFF, TAIL), pl.ds(c0, DH)],
                    rows_v.at[0, pl.ds(0, TAIL)])
    pltpu.sync_copy(rows_v.at[0], acc_sh.at[ids_v.at[0]], add=True)
    pltpu.sync_copy(ones_v, cnt_sh.at[ids_v.at[0]], add=True)

  plsc.subcore_barrier()

  g0 = tid * GPT
  pltpu.sync_copy(acc_sh.at[pl.ds(g0, GPT)], rows_v.at[0, pl.ds(0, GPT)])
  pltpu.sync_copy(cnt_sh.at[pl.ds(g0, GPT)], cnt_v)

  def div_body(r, _):
    cnt = cnt_v[r, pl.ds(0, L)]
    for k in range(DH // L):
      rows_v[0, r, pl.ds(k * L, L)] = rows_v[0, r, pl.ds(k * L, L)] / cnt
    return 0
  lax.fori_loop(0, GPT, div_body, 0)

  pltpu.sync_copy(rows_v.at[0, pl.ds(0, GPT)],
                  out_hbm.at[pl.ds(g0, GPT), pl.ds(c0, DH)])


def kernel(x, batch):
  mesh = plsc.VectorSubcoreMesh(core_axis_name="c", subcore_axis_name="s")
  return pl.kernel(
      _seg_mean_kernel,
      out_type=jax.ShapeDtypeStruct((G, D), jnp.float32),
      mesh=mesh,
      scratch_types=[
          pltpu.VMEM_SHARED((ACC_ROWS, DH), jnp.float32),
          pltpu.VMEM_SHARED((ACC_ROWS, L), jnp.float32),
          pltpu.VMEM((2, CH, DH), jnp.float32),
          pltpu.VMEM((2, CH), jnp.int32),
          pltpu.VMEM((CH, L), jnp.float32),
          pltpu.VMEM((GPT, L), jnp.float32),
          pltpu.SemaphoreType.DMA,
          pltpu.SemaphoreType.DMA,
      ],
      compiler_params=pltpu.CompilerParams(use_tc_tiling_on_sc=False),
  )(x, batch)


# speedup vs baseline: 6.0620x; 1.0108x over previous
import jax
import jax.numpy as jnp
from jax import lax
from jax.experimental import pallas as pl
from jax.experimental.pallas import tpu as pltpu
from jax.experimental.pallas import tpu_sc as plsc

N = 100000
D = 128
G = 256

NC = 2
NS = 16
L = 16
DH = D // NC
CH = 128
NBUF = 4

NFULL = N // CH
TAIL = N - NFULL * CH
TAIL_OFF = NFULL * CH
CPT = 49
LPT = NFULL - (NS - 1) * CPT
ACC_ROWS = G + NS
ZR = ACC_ROWS // NS
GPT = G // NS


def _seg_mean_kernel(x_hbm, b2d_hbm, btail_hbm, out_hbm,
                     acc_sh, cnt_sh, rows_v, ids_v, ones_v, cnt_v,
                     sem_i, sem0, sem1, sem2, sem3):
  tid = lax.axis_index("s")
  cid = lax.axis_index("c")
  c0 = cid * DH
  dummy = (G + tid).astype(jnp.int32)
  chunk0 = tid * CPT
  n_chunks = jnp.where(tid < NS - 1, CPT, LPT)
  sems = (sem0, sem1, sem2, sem3)

  @pl.when(tid < NS - 1)
  def _ids_full():
    pltpu.async_copy(b2d_hbm.at[pl.ds(chunk0, CPT)], ids_v.at[pl.ds(0, CPT)],
                     sem_i)

  @pl.when(tid == NS - 1)
  def _ids_last():
    pltpu.async_copy(b2d_hbm.at[pl.ds(chunk0, LPT)], ids_v.at[pl.ds(0, LPT)],
                     sem_i)

  def zero_body(r, _):
    for k in range(DH // L):
      rows_v[0, r, pl.ds(k * L, L)] = jnp.zeros((L,), jnp.float32)
    ones_v[r, pl.ds(0, L)] = jnp.zeros((L,), jnp.float32)
    return 0
  lax.fori_loop(0, ZR, zero_body, 0)
  pltpu.sync_copy(rows_v.at[0, pl.ds(0, ZR)], acc_sh.at[pl.ds(tid * ZR, ZR)])
  pltpu.sync_copy(ones_v.at[pl.ds(0, ZR)], cnt_sh.at[pl.ds(tid * ZR, ZR)])

  def ones_body(r, _):
    ones_v[r, pl.ds(0, L)] = jnp.ones((L,), jnp.float32)
    return 0
  lax.fori_loop(0, CH, ones_body, 0)

  @pl.when(tid < NS - 1)
  def _ids_full_wait():
    pltpu.make_async_copy(b2d_hbm.at[pl.ds(chunk0, CPT)],
                          ids_v.at[pl.ds(0, CPT)], sem_i).wait()

  @pl.when(tid == NS - 1)
  def _ids_last_wait():
    pltpu.make_async_copy(b2d_hbm.at[pl.ds(chunk0, LPT)],
                          ids_v.at[pl.ds(0, LPT)], sem_i).wait()

  plsc.subcore_barrier()

  def row_copy(k, b):
    off = (chunk0 + k) * CH
    return pltpu.make_async_copy(x_hbm.at[pl.ds(off, CH), pl.ds(c0, DH)],
                                 rows_v.at[b], sems[b])

  for b in range(NBUF):
    row_copy(b, b).start()

  def ring_body(i, _):
    for b in range(NBUF):
      k = NBUF * i + b

      @pl.when(k < n_chunks)
      def _process():
        row_copy(k, b).wait()
        pltpu.sync_copy(rows_v.at[b], acc_sh.at[ids_v.at[k]], add=True)
        pltpu.sync_copy(ones_v, cnt_sh.at[ids_v.at[k]], add=True)

        @pl.when(k + NBUF < n_chunks)
        def _prefetch():
          row_copy(k + NBUF, b).start()
    return 0
  lax.fori_loop(0, (n_chunks + NBUF - 1) // NBUF, ring_body, 0)

  @pl.when(tid == NS - 1)
  def _tail():
    for r in range(CH // L):
      ids_v[LPT, pl.ds(r * L, L)] = jnp.full((L,), 0, jnp.int32) + dummy
    pltpu.sync_copy(btail_hbm.at[pl.ds(0, TAIL)], ids_v.at[LPT, pl.ds(0, TAIL)])
    pltpu.sync_copy(x_hbm.at[pl.ds(TAIL_OFF, TAIL), pl.ds(c0, DH)],
                    rows_v.at[0, pl.ds(0, TAIL)])
    pltpu.sync_copy(rows_v.at[0], acc_sh.at[ids_v.at[LPT]], add=True)
    pltpu.sync_copy(ones_v, cnt_sh.at[ids_v.at[LPT]], add=True)

  plsc.subcore_barrier()

  g0 = tid * GPT
  pltpu.sync_copy(acc_sh.at[pl.ds(g0, GPT)], rows_v.at[0, pl.ds(0, GPT)])
  pltpu.sync_copy(cnt_sh.at[pl.ds(g0, GPT)], cnt_v)

  def div_body(r, _):
    cnt = cnt_v[r, pl.ds(0, L)]
    for k in range(DH // L):
      rows_v[0, r, pl.ds(k * L, L)] = rows_v[0, r, pl.ds(k * L, L)] / cnt
    return 0
  lax.fori_loop(0, GPT, div_body, 0)

  pltpu.sync_copy(rows_v.at[0, pl.ds(0, GPT)],
                  out_hbm.at[pl.ds(g0, GPT), pl.ds(c0, DH)])


def kernel(x, batch):
  b2d = batch[:TAIL_OFF].reshape(NFULL, CH)
  btail = batch[TAIL_OFF:]
  mesh = plsc.VectorSubcoreMesh(core_axis_name="c", subcore_axis_name="s")
  return pl.kernel(
      _seg_mean_kernel,
      out_type=jax.ShapeDtypeStruct((G, D), jnp.float32),
      mesh=mesh,
      scratch_types=[
          pltpu.VMEM_SHARED((ACC_ROWS, DH), jnp.float32),
          pltpu.VMEM_SHARED((ACC_ROWS, L), jnp.float32),
          pltpu.VMEM((NBUF, CH, DH), jnp.float32),
          pltpu.VMEM((CPT + 1, CH), jnp.int32),
          pltpu.VMEM((CH, L), jnp.float32),
          pltpu.VMEM((GPT, L), jnp.float32),
          pltpu.SemaphoreType.DMA,
          pltpu.SemaphoreType.DMA,
          pltpu.SemaphoreType.DMA,
          pltpu.SemaphoreType.DMA,
          pltpu.SemaphoreType.DMA,
      ],
      compiler_params=pltpu.CompilerParams(use_tc_tiling_on_sc=False),
  )(x, b2d, btail)


# speedup vs baseline: 6.1141x; 1.0086x over previous
import jax
import jax.numpy as jnp
from jax import lax
from jax.experimental import pallas as pl
from jax.experimental.pallas import tpu as pltpu
from jax.experimental.pallas import tpu_sc as plsc

N = 100000
D = 128
G = 256

NC = 2
NS = 16
L = 16
DH = D // NC
CH = 128
NBUF = 6
PF = 3

NFULL = N // CH
TAIL = N - NFULL * CH
TAIL_OFF = NFULL * CH
CPT = 49
LPT = NFULL - (NS - 1) * CPT
ACC_ROWS = G + NS
ZR = ACC_ROWS // NS
GPT = G // NS


def _seg_mean_kernel(x_hbm, b2d_hbm, btail_hbm, out_hbm,
                     acc_sh, cnt_sh, rows_v, ids_v, ones_v, cnt_v,
                     sem_i, sem0, sem1, sem2, sem3, sem4, sem5,
                     ssem0, ssem1, ssem2, ssem3, ssem4, ssem5):
  tid = lax.axis_index("s")
  cid = lax.axis_index("c")
  c0 = cid * DH
  dummy = (G + tid).astype(jnp.int32)
  chunk0 = tid * CPT
  n_chunks = jnp.where(tid < NS - 1, CPT, LPT)
  sems = (sem0, sem1, sem2, sem3, sem4, sem5)
  ssems = (ssem0, ssem1, ssem2, ssem3, ssem4, ssem5)

  @pl.when(tid < NS - 1)
  def _ids_full():
    pltpu.async_copy(b2d_hbm.at[pl.ds(chunk0, CPT)], ids_v.at[pl.ds(0, CPT)],
                     sem_i)

  @pl.when(tid == NS - 1)
  def _ids_last():
    pltpu.async_copy(b2d_hbm.at[pl.ds(chunk0, LPT)], ids_v.at[pl.ds(0, LPT)],
                     sem_i)

  def zero_body(r, _):
    for k in range(DH // L):
      rows_v[0, r, pl.ds(k * L, L)] = jnp.zeros((L,), jnp.float32)
    ones_v[r, pl.ds(0, L)] = jnp.zeros((L,), jnp.float32)
    return 0
  lax.fori_loop(0, ZR, zero_body, 0)
  pltpu.sync_copy(rows_v.at[0, pl.ds(0, ZR)], acc_sh.at[pl.ds(tid * ZR, ZR)])
  pltpu.sync_copy(ones_v.at[pl.ds(0, ZR)], cnt_sh.at[pl.ds(tid * ZR, ZR)])

  def ones_body(r, _):
    ones_v[r, pl.ds(0, L)] = jnp.ones((L,), jnp.float32)
    return 0
  lax.fori_loop(0, CH, ones_body, 0)

  @pl.when(tid < NS - 1)
  def _ids_full_wait():
    pltpu.make_async_copy(b2d_hbm.at[pl.ds(chunk0, CPT)],
                          ids_v.at[pl.ds(0, CPT)], sem_i).wait()

  @pl.when(tid == NS - 1)
  def _ids_last_wait():
    pltpu.make_async_copy(b2d_hbm.at[pl.ds(chunk0, LPT)],
                          ids_v.at[pl.ds(0, LPT)], sem_i).wait()

  plsc.subcore_barrier()

  def row_copy(k, b):
    off = (chunk0 + k) * CH
    return pltpu.make_async_copy(x_hbm.at[pl.ds(off, CH), pl.ds(c0, DH)],
                                 rows_v.at[b], sems[b])

  def scat_desc(k, b):
    return pltpu.make_async_copy(rows_v.at[b], acc_sh.at[ids_v.at[k]],
                                 ssems[b])

  for b in range(PF):
    row_copy(b, b).start()

  def ring_body(i, _):
    for b in range(NBUF):
      k = NBUF * i + b

      @pl.when(k < n_chunks)
      def _process():
        row_copy(k, b).wait()
        pltpu.async_copy(rows_v.at[b], acc_sh.at[ids_v.at[k]], ssems[b],
                         add=True)
        pltpu.sync_copy(ones_v, cnt_sh.at[ids_v.at[k]], add=True)
        j = k + PF
        bj = (b + PF) % NBUF

        @pl.when(j < n_chunks)
        def _prefetch():
          @pl.when(j >= NBUF)
          def _free_slot():
            scat_desc(j - NBUF, bj).wait()
          row_copy(j, bj).start()
    return 0
  lax.fori_loop(0, (n_chunks + NBUF - 1) // NBUF, ring_body, 0)

  for b in range(NBUF):
    scat_desc(0, b).wait()

  @pl.when(tid == NS - 1)
  def _tail():
    for r in range(CH // L):
      ids_v[LPT, pl.ds(r * L, L)] = jnp.full((L,), 0, jnp.int32) + dummy
    pltpu.sync_copy(btail_hbm.at[pl.ds(0, TAIL)], ids_v.at[LPT, pl.ds(0, TAIL)])
    pltpu.sync_copy(x_hbm.at[pl.ds(TAIL_OFF, TAIL), pl.ds(c0, DH)],
                    rows_v.at[0, pl.ds(0, TAIL)])
    pltpu.sync_copy(rows_v.at[0], acc_sh.at[ids_v.at[LPT]], add=True)
    pltpu.sync_copy(ones_v, cnt_sh.at[ids_v.at[LPT]], add=True)

  plsc.subcore_barrier()

  g0 = tid * GPT
  pltpu.sync_copy(acc_sh.at[pl.ds(g0, GPT)], rows_v.at[0, pl.ds(0, GPT)])
  pltpu.sync_copy(cnt_sh.at[pl.ds(g0, GPT)], cnt_v)

  def div_body(r, _):
    cnt = cnt_v[r, pl.ds(0, L)]
    for k in range(DH // L):
      rows_v[0, r, pl.ds(k * L, L)] = rows_v[0, r, pl.ds(k * L, L)] / cnt
    return 0
  lax.fori_loop(0, GPT, div_body, 0)

  pltpu.sync_copy(rows_v.at[0, pl.ds(0, GPT)],
                  out_hbm.at[pl.ds(g0, GPT), pl.ds(c0, DH)])


def kernel(x, batch):
  b2d = batch[:TAIL_OFF].reshape(NFULL, CH)
  btail = batch[TAIL_OFF:]
  mesh = plsc.VectorSubcoreMesh(core_axis_name="c", subcore_axis_name="s")
  return pl.kernel(
      _seg_mean_kernel,
      out_type=jax.ShapeDtypeStruct((G, D), jnp.float32),
      mesh=mesh,
      scratch_types=[
          pltpu.VMEM_SHARED((ACC_ROWS, DH), jnp.float32),
          pltpu.VMEM_SHARED((ACC_ROWS, L), jnp.float32),
          pltpu.VMEM((NBUF, CH, DH), jnp.float32),
          pltpu.VMEM((CPT + 1, CH), jnp.int32),
          pltpu.VMEM((CH, L), jnp.float32),
          pltpu.VMEM((GPT, L), jnp.float32),
          pltpu.SemaphoreType.DMA,
          pltpu.SemaphoreType.DMA,
          pltpu.SemaphoreType.DMA,
          pltpu.SemaphoreType.DMA,
          pltpu.SemaphoreType.DMA,
          pltpu.SemaphoreType.DMA,
          pltpu.SemaphoreType.DMA,
          pltpu.SemaphoreType.DMA,
          pltpu.SemaphoreType.DMA,
          pltpu.SemaphoreType.DMA,
          pltpu.SemaphoreType.DMA,
          pltpu.SemaphoreType.DMA,
          pltpu.SemaphoreType.DMA,
      ],
      compiler_params=pltpu.CompilerParams(use_tc_tiling_on_sc=False),
  )(x, b2d, btail)
